# baseline (speedup 1.0000x reference)
import functools

import jax
import jax.numpy as jnp
from jax import lax
from jax.experimental import pallas as pl
from jax.experimental.pallas import tpu as pltpu
from jax.experimental.pallas import tpu_sc as plsc

B, M, S, C, V, D = 32, 50, 20, 1000, 100000, 64
HOPS = 3
NC, NS = 2, 16
NW = NC * NS
VGRID = 4
VB = V // VGRID
CCH = 16
NCH = 63
CPAD = NCH * CCH

_mesh = plsc.VectorSubcoreMesh(
    core_axis_name="c", subcore_axis_name="s", num_cores=NC, num_subcores=NS)
_sc_params = pltpu.CompilerParams(
    use_tc_tiling_on_sc=False, needs_layout_passes=False)


def _wid():
  return lax.axis_index("s") * NC + lax.axis_index("c")


@functools.partial(
    pl.kernel,
    out_type=(
        jax.ShapeDtypeStruct((B, M, D), jnp.float32),
        jax.ShapeDtypeStruct((B, D), jnp.float32),
    ),
    mesh=_mesh,
    compiler_params=_sc_params,
    scratch_types=[
        pltpu.VMEM((M * S,), jnp.int32),
        pltpu.VMEM((S,), jnp.int32),
        pltpu.VMEM((M * S + S, D), jnp.float32),
        pltpu.VMEM((M + 1, D), jnp.float32),
        pltpu.SemaphoreType.DMA,
    ],
)
def _sc_gather(stories_hbm, query_hbm, A_hbm,
               es_hbm, u0_hbm, sv, qv, rows_v, oa_v, sem):
  w = _wid()
  pltpu.sync_copy(stories_hbm.at[w], sv)
  pltpu.sync_copy(query_hbm.at[w], qv)

  a_chunks = [(c * 4, 4) for c in range(M // 4)] + [(M - 2, 2)]
  cps = [
      pltpu.async_copy(A_hbm.at[sv.at[pl.ds(s0 * S, n * S)]],
                       rows_v.at[pl.ds(s0 * S, n * S)], sem)
      for s0, n in a_chunks
  ]
  qcp = pltpu.async_copy(A_hbm.at[qv], rows_v.at[pl.ds(M * S, S)], sem)
  for cp in cps:
    cp.wait()
  qcp.wait()

  def seg_body(i, _):
    base = i * S
    for d in range(D // 16):
      sl = pl.ds(d * 16, 16)
      acc = rows_v[base, sl]
      for r in range(1, S):
        acc = acc + rows_v[base + r, sl]
      oa_v[i, sl] = acc
    return 0

  lax.fori_loop(0, M + 1, seg_body, 0)
  pltpu.sync_copy(oa_v.at[pl.ds(0, M)], es_hbm.at[w])
  pltpu.sync_copy(oa_v.at[M], u0_hbm.at[w])


def _tc_body(u0_ref, es_ref, Hw_ref, Hb_ref, w_ref, proj_ref, u_sc):
  i = pl.program_id(0)

  @pl.when(i == 0)
  def _():
    u = u0_ref[...]
    es = es_ref[...]
    Hw = Hw_ref[...]
    Hb = Hb_ref[...]
    for _ in range(HOPS):
      sc = jnp.sum(es * u[:, None, :], axis=2)
      sc = sc - jnp.max(sc, axis=1, keepdims=True)
      e = jnp.exp(sc)
      att = e / jnp.sum(e, axis=1, keepdims=True)
      attn = jnp.sum(att[:, :, None] * es, axis=1)
      u = lax.dot_general(u, Hw, (((1,), (1,)), ((), ()))) + Hb + attn
    u_sc[...] = u

  proj_ref[0] = lax.dot_general(u_sc[...], w_ref[...],
                                (((1,), (1,)), ((), ())))


_tc_stage = pl.pallas_call(
    _tc_body,
    grid=(VGRID,),
    in_specs=[
        pl.BlockSpec((B, D), lambda i: (0, 0)),
        pl.BlockSpec((B, M, D), lambda i: (0, 0, 0)),
        pl.BlockSpec((D, D), lambda i: (0, 0)),
        pl.BlockSpec((1, D), lambda i: (0, 0)),
        pl.BlockSpec((VB, D), lambda i: (i, 0)),
    ],
    out_specs=pl.BlockSpec((1, B, VB), lambda i: (i, 0, 0)),
    out_shape=jax.ShapeDtypeStruct((VGRID, B, VB), jnp.float32),
    scratch_shapes=[pltpu.VMEM((B, D), jnp.float32)],
)


@functools.partial(
    pl.kernel,
    out_type=jax.ShapeDtypeStruct((B, C), jnp.float32),
    mesh=_mesh,
    compiler_params=_sc_params,
    scratch_types=[
        pltpu.VMEM((V,), jnp.float32),
        pltpu.VMEM((CPAD * S,), jnp.int32),
        pltpu.VMEM((CPAD,), jnp.float32),
        pltpu.SemaphoreType.DMA,
    ],
)
def _sc_score(proj_hbm, E_hbm, cand_hbm, out_hbm, projv, ev, outv, sem):
  b = _wid()
  pcps = [
      pltpu.async_copy(proj_hbm.at[k, b], projv.at[pl.ds(k * VB, VB)], sem)
      for k in range(VGRID)
  ]
  ecp = pltpu.async_copy(E_hbm.at[b], ev.at[pl.ds(0, C * S)], sem)

  izeros = jnp.zeros((16,), jnp.int32)
  for z in range(C * S // 16, CPAD * S // 16):
    ev[pl.ds(z * 16, 16)] = izeros
  fzeros = jnp.zeros((16,), jnp.float32)
  for cc in range(NCH):
    outv[pl.ds(cc * 16, 16)] = fzeros
  lanes = lax.iota(jnp.int32, 16) * S

  for cp in pcps:
    cp.wait()
  ecp.wait()

  def gather_phase():
    def s_body(s, _):
      for cc in range(NCH):
        pos = lanes + (cc * (CCH * S) + s)
        idx = plsc.load_gather(ev, [pos])
        vals = plsc.load_gather(projv, [idx])
        sl = pl.ds(cc * 16, 16)
        outv[sl] = outv[sl] + vals
      return 0
    lax.fori_loop(0, S, s_body, 0)

  gather_phase()
  pltpu.sync_copy(cand_hbm, ev.at[pl.ds(0, C * S)])
  gather_phase()
  pltpu.sync_copy(outv.at[pl.ds(0, C)], out_hbm.at[b])


def kernel(stories, query, E, candidates, A, W, H_w, H_b):
  stories = stories.astype(jnp.int32)
  query = query.astype(jnp.int32)
  E = E.astype(jnp.int32)
  candidates = candidates.astype(jnp.int32)

  es, u0 = _sc_gather(stories.reshape(B, M * S), query, A)
  proj = _tc_stage(u0, es, H_w, H_b.reshape(1, D), W)
  return _sc_score(proj, E.reshape(B, C * S), candidates.reshape(C * S))
